# baseline (speedup 1.0000x reference)
import jax
import jax.numpy as jnp
import numpy as np
from jax.experimental import pallas as pl
from jax.experimental.pallas import tpu as pltpu

B, S, D, DC, NS, NL, LI, NW, KS = 2, 1024, 1024, 64, 1024, 8, 3, 8, 5
NTOT = NL * NS
ISQ = float(1.0 / np.sqrt(DC))
TS = 512
DTILE = 256
DT = D // DTILE
BF = jnp.bfloat16


def _shift_store(hnc_ref, hn, dil):
    for k in range(KS):
        shift = (KS - 1 - k) * dil
        if shift:
            sh = jnp.concatenate(
                [jnp.zeros((shift, D), BF), hn[:S - shift]], axis=0)
        else:
            sh = hn
        hnc_ref[:, k * D:(k + 1) * D] = sh


def _ln(x, g, b):
    m = jnp.mean(x, axis=1, keepdims=True)
    v = jnp.mean((x - m) ** 2, axis=1, keepdims=True)
    return (x - m) * jax.lax.rsqrt(v + 1e-5) * g + b


def _mega_a(x_ref, cache_ref, wmc_ref, wmv_ref, wq_ref, wo_ref, wgr_ref,
            wqw_ref, wow_ref, wgwr_ref, w0_ref, b0_ref, g0_ref, be0_ref,
            h1_ref, cba_ref, xet_ref, hnf_ref, hnc_ref):
    dt = pl.program_id(1)

    @pl.when(dt == 0)
    def _():
        cba_ref[:, 0:DC] = cache_ref[0].astype(BF)
        il = jax.lax.broadcasted_iota(jnp.int32, (NTOT, DC), 1)
        cba_ref[:, DC:2 * DC] = jnp.where(il == 0, 1.0, 0.0).astype(BF)
        cb = cba_ref[:, 0:DC]
        content = wmc_ref[0]
        contentb = content.astype(BF)
        logv = jnp.log(wmv_ref[0] + 1e-6)

        for st in range(S // TS):
            x = x_ref[0, st * TS:(st + 1) * TS, :]
            xb = x.astype(BF)
            q = jnp.dot(xb, wq_ref[...].astype(BF),
                        preferred_element_type=jnp.float32)
            qb = (q * ISQ).astype(BF)
            logits = jax.lax.dot_general(
                qb, cb, (((1,), (1,)), ((), ())),
                preferred_element_type=jnp.float32).astype(BF)
            m = jnp.max(logits, axis=1, keepdims=True)
            e = jnp.exp(logits - m)
            ra = jnp.dot(e, cba_ref[...], preferred_element_type=jnp.float32)
            read = ra[:, 0:DC] / ra[:, DC:DC + 1]
            g = jax.nn.sigmoid(jnp.dot(xb, wgr_ref[...].astype(BF),
                                       preferred_element_type=jnp.float32))
            x_ltm = x + g * jnp.dot(read.astype(BF), wo_ref[...].astype(BF),
                                    preferred_element_type=jnp.float32)

            xlb = x_ltm.astype(BF)
            qw = jnp.dot(xlb, wqw_ref[...].astype(BF),
                         preferred_element_type=jnp.float32)
            sw = jax.lax.dot_general((qw * ISQ).astype(BF), contentb,
                                     (((1,), (1,)), ((), ())),
                                     preferred_element_type=jnp.float32)
            sw = sw + logv
            mw = jnp.max(sw, axis=1, keepdims=True)
            ew = jnp.exp(sw - mw)
            aw = (ew / jnp.sum(ew, axis=1, keepdims=True)).astype(BF)
            readw = jnp.dot(aw, contentb, preferred_element_type=jnp.float32)
            gw = jax.nn.sigmoid(jnp.dot(xlb, wgwr_ref[...].astype(BF),
                                        preferred_element_type=jnp.float32))
            xe = x_ltm + gw * jnp.dot(readw.astype(BF), wow_ref[...].astype(BF),
                                      preferred_element_type=jnp.float32)

            hnf_ref[st * TS:(st + 1) * TS, :] = _ln(
                xe, g0_ref[...], be0_ref[...]).astype(BF)
            for j in range(DT):
                xet_ref[j, st * TS:(st + 1) * TS, :] = \
                    xe[:, j * DTILE:(j + 1) * DTILE].astype(BF)

        _shift_store(hnc_ref, hnf_ref[...], 1)

    part = jnp.dot(hnc_ref[...], w0_ref[...].astype(BF),
                   preferred_element_type=jnp.float32)
    h1_ref[0] = (xet_ref[dt].astype(jnp.float32)
                 + jax.nn.gelu(part + b0_ref[...])).astype(BF)


def _mega_b(h1f_ref, lc_ref, wmc_ref, wmv_ref, w1_ref, b1_ref,
            g1_ref, be1_ref, png_ref, pnb_ref, wqww_ref, wvww_ref, wgww_ref,
            wk_ref, wv_ref, wg_ref,
            out_ref, slice_ref, wmc_out_ref, wmv_out_ref,
            hnc_ref, h1tt_ref, h2t_ref):
    dt = pl.program_id(1)

    @pl.when(dt == 0)
    def _():
        h1 = h1f_ref[0].astype(jnp.float32)
        hn = _ln(h1, g1_ref[...], be1_ref[...]).astype(BF)
        _shift_store(hnc_ref, hn, 2)
        for j in range(DT):
            h1tt_ref[j] = h1f_ref[0][:, j * DTILE:(j + 1) * DTILE]

    part = jnp.dot(hnc_ref[...], w1_ref[...].astype(BF),
                   preferred_element_type=jnp.float32)
    h2t_ref[dt] = h1tt_ref[dt].astype(jnp.float32) + jax.nn.gelu(part + b1_ref[...])

    @pl.when(dt == DT - 1)
    def _():
        h2 = jnp.concatenate([h2t_ref[j] for j in range(DT)], axis=1)
        o = _ln(h2, png_ref[...], pnb_ref[...])
        out_ref[0] = o
        pooled = jnp.mean(o, axis=0, keepdims=True)

        content = wmc_ref[0]
        valid = wmv_ref[0]
        pq = jnp.dot(pooled, wqww_ref[...], preferred_element_type=jnp.float32)
        ws = jax.lax.dot_general(pq, content, (((1,), (1,)), ((), ())),
                                 preferred_element_type=jnp.float32)
        mx = jnp.max(ws, axis=1, keepdims=True)
        iota_l = jax.lax.broadcasted_iota(jnp.int32, (1, NW), 1)
        slot = jnp.min(jnp.where(ws >= mx, iota_l, NW))
        mask_col = jax.lax.broadcasted_iota(jnp.int32, (NW, 1), 0) == slot
        wv_val = jnp.dot(pooled, wvww_ref[...],
                         preferred_element_type=jnp.float32)
        wg_val = jax.nn.sigmoid(jnp.dot(pooled, wgww_ref[...],
                                        preferred_element_type=jnp.float32))
        old = jnp.sum(jnp.where(mask_col, content, 0.0), axis=0, keepdims=True)
        newc = wg_val * wv_val + (1.0 - wg_val) * old
        wmc_out_ref[0] = jnp.where(mask_col, newc, content)
        wgs = wg_val[0, 0]
        wmv_out_ref[0] = jnp.where(iota_l == slot,
                                   jnp.maximum(valid, wgs), valid)

        ob = o.astype(BF)
        lc = lc_ref[0]
        lcb = lc.astype(BF)
        kx = jnp.dot(ob, wk_ref[...].astype(BF),
                     preferred_element_type=jnp.float32)
        vx = jnp.dot(ob, wv_ref[...].astype(BF),
                     preferred_element_type=jnp.float32)
        al = jax.lax.dot_general((kx * ISQ).astype(BF), lcb,
                                 (((1,), (1,)), ((), ())),
                                 preferred_element_type=jnp.float32).astype(BF)
        m = jnp.max(al, axis=1, keepdims=True)
        e = jnp.exp(al - m)
        rs = jnp.sum(e.astype(jnp.float32), axis=1, keepdims=True)
        gw = jax.nn.sigmoid(jnp.dot(ob, wg_ref[...].astype(BF),
                                    preferred_element_type=jnp.float32))
        wts = e * (gw / rs).astype(BF)
        ones = jnp.ones((S, 1), BF)
        wsum = jax.lax.dot_general(wts, ones, (((0,), (0,)), ((), ())),
                                   preferred_element_type=jnp.float32)
        vavg = jax.lax.dot_general(wts, vx.astype(BF), (((0,), (0,)), ((), ())),
                                   preferred_element_type=jnp.float32)
        vavg = vavg / (wsum + 1e-6)
        blend = jnp.clip(wsum, 0.0, 1.0)
        slice_ref[0] = lc * (1.0 - blend) + vavg * blend


def _row2d(a):
    return a.reshape(1, -1)


def kernel(x, cache, wm, Wq_ltm, Wo_ltm, Wg_ltm_r, Wk_ltm_w, Wv_ltm_w,
           Wg_ltm_w, Wq_wm, Wo_wm, Wg_wm_r, Wq_wm_w, Wv_wm_w, Wg_wm_w,
           conv0_w, conv1_w, conv0_b, conv1_b, ln0_g, ln0_b, ln1_g, ln1_b,
           pn_g, pn_b):
    wmc = wm[..., :DC]
    wmv = jnp.transpose(wm[..., DC:], (0, 2, 1))
    w0 = conv0_w.reshape(KS * D, D)
    w1 = conv1_w.reshape(KS * D, D)

    full = lambda *shape: pl.BlockSpec(shape, lambda b, dt: (0,) * len(shape))

    h1 = pl.pallas_call(
        _mega_a,
        grid=(B, DT),
        in_specs=[
            pl.BlockSpec((1, S, D), lambda b, dt: (b, 0, 0)),
            pl.BlockSpec((1, NTOT, DC), lambda b, dt: (b, 0, 0)),
            pl.BlockSpec((1, NW, DC), lambda b, dt: (b, 0, 0)),
            pl.BlockSpec((1, 1, NW), lambda b, dt: (b, 0, 0)),
            full(D, DC), full(DC, D), full(D, 1),
            full(D, DC), full(DC, D), full(D, 1),
            pl.BlockSpec((KS * D, DTILE), lambda b, dt: (0, dt)),
            pl.BlockSpec((1, DTILE), lambda b, dt: (0, dt)),
            full(1, D), full(1, D),
        ],
        out_specs=pl.BlockSpec((1, S, DTILE), lambda b, dt: (b, 0, dt)),
        out_shape=jax.ShapeDtypeStruct((B, S, D), BF),
        scratch_shapes=[
            pltpu.VMEM((NTOT, 2 * DC), BF),
            pltpu.VMEM((DT, S, DTILE), BF),
            pltpu.VMEM((S, D), BF),
            pltpu.VMEM((S, KS * D), BF),
        ],
        compiler_params=pltpu.CompilerParams(
            dimension_semantics=("parallel", "arbitrary")),
    )(x, cache, wmc, wmv, Wq_ltm, Wo_ltm, Wg_ltm_r, Wq_wm, Wo_wm, Wg_wm_r,
      w0, _row2d(conv0_b), _row2d(ln0_g), _row2d(ln0_b))

    output, new_slice, wmc_u, wmv_u = pl.pallas_call(
        _mega_b,
        grid=(B, DT),
        in_specs=[
            pl.BlockSpec((1, S, D), lambda b, dt: (b, 0, 0)),
            pl.BlockSpec((1, NS, DC), lambda b, dt: (b, LI, 0)),
            pl.BlockSpec((1, NW, DC), lambda b, dt: (b, 0, 0)),
            pl.BlockSpec((1, 1, NW), lambda b, dt: (b, 0, 0)),
            pl.BlockSpec((KS * D, DTILE), lambda b, dt: (0, dt)),
            pl.BlockSpec((1, DTILE), lambda b, dt: (0, dt)),
            full(1, D), full(1, D), full(1, D), full(1, D),
            full(D, DC), full(D, DC), full(D, 1),
            full(D, DC), full(D, DC), full(D, 1),
        ],
        out_specs=[
            pl.BlockSpec((1, S, D), lambda b, dt: (b, 0, 0)),
            pl.BlockSpec((1, NS, DC), lambda b, dt: (b, 0, 0)),
            pl.BlockSpec((1, NW, DC), lambda b, dt: (b, 0, 0)),
            pl.BlockSpec((1, 1, NW), lambda b, dt: (b, 0, 0)),
        ],
        out_shape=[
            jax.ShapeDtypeStruct((B, S, D), jnp.float32),
            jax.ShapeDtypeStruct((B, NS, DC), jnp.float32),
            jax.ShapeDtypeStruct((B, NW, DC), jnp.float32),
            jax.ShapeDtypeStruct((B, 1, NW), jnp.float32),
        ],
        scratch_shapes=[
            pltpu.VMEM((S, KS * D), BF),
            pltpu.VMEM((DT, S, DTILE), BF),
            pltpu.VMEM((DT, S, DTILE), jnp.float32),
        ],
        compiler_params=pltpu.CompilerParams(
            dimension_semantics=("parallel", "arbitrary")),
    )(h1, cache, wmc, wmv, w1, _row2d(conv1_b), _row2d(ln1_g),
      _row2d(ln1_b), _row2d(pn_g), _row2d(pn_b), Wq_wm_w, Wv_wm_w, Wg_wm_w,
      Wk_ltm_w, Wv_ltm_w, Wg_ltm_w)

    cache_u = jax.lax.dynamic_update_slice_in_dim(cache, new_slice,
                                                  LI * NS, axis=1)
    wm_u = jnp.concatenate([wmc_u, jnp.transpose(wmv_u, (0, 2, 1))], axis=-1)
    return (output, cache_u, wm_u)


# speedup vs baseline: 1.0708x; 1.0708x over previous
import jax
import jax.numpy as jnp
import numpy as np
from jax.experimental import pallas as pl
from jax.experimental.pallas import tpu as pltpu

B, S, D, DC, NS, NL, LI, NW, KS = 2, 1024, 1024, 64, 1024, 8, 3, 8, 5
NTOT = NL * NS
ISQ = float(1.0 / np.sqrt(DC))
TS = 256
DTILE = 256
DT = D // DTILE
BF = jnp.bfloat16


def _shift_store(hnc_ref, hn, dil):
    for k in range(KS):
        shift = (KS - 1 - k) * dil
        if shift:
            sh = jnp.concatenate(
                [jnp.zeros((shift, D), BF), hn[:S - shift]], axis=0)
        else:
            sh = hn
        hnc_ref[:, k * D:(k + 1) * D] = sh


def _ln(x, g, b):
    m = jnp.mean(x, axis=1, keepdims=True)
    v = jnp.mean((x - m) ** 2, axis=1, keepdims=True)
    return (x - m) * jax.lax.rsqrt(v + 1e-5) * g + b


def _mega_a(x_ref, cache_ref, wmc_ref, wmv_ref, wq_ref, wo_ref, wgr_ref,
            wqw_ref, wow_ref, wgwr_ref, w0_ref, b0_ref, g0_ref, be0_ref,
            h1_ref, cba_ref, xet_ref, hnf_ref, hnc_ref):
    dt = pl.program_id(1)

    @pl.when(dt == 0)
    def _():
        cba_ref[:, 0:DC] = cache_ref[0].astype(BF)
        il = jax.lax.broadcasted_iota(jnp.int32, (NTOT, DC), 1)
        cba_ref[:, DC:2 * DC] = jnp.where(il == 0, 1.0, 0.0).astype(BF)
        cb = cba_ref[:, 0:DC]
        content = wmc_ref[0]
        contentb = content.astype(BF)
        logv = jnp.log(wmv_ref[0] + 1e-6)

        for st in range(S // TS):
            x = x_ref[0, st * TS:(st + 1) * TS, :]
            xb = x.astype(BF)
            q = jnp.dot(xb, wq_ref[...].astype(BF),
                        preferred_element_type=jnp.float32)
            qb = (q * ISQ).astype(BF)
            logits = jax.lax.dot_general(
                qb, cb, (((1,), (1,)), ((), ())),
                preferred_element_type=jnp.float32).astype(BF)
            m = jnp.max(logits, axis=1, keepdims=True)
            e = jnp.exp(logits - m)
            ra = jnp.dot(e, cba_ref[...], preferred_element_type=jnp.float32)
            read = ra[:, 0:DC] / ra[:, DC:DC + 1]
            g = jax.nn.sigmoid(jnp.dot(xb, wgr_ref[...].astype(BF),
                                       preferred_element_type=jnp.float32))
            x_ltm = x + g * jnp.dot(read.astype(BF), wo_ref[...].astype(BF),
                                    preferred_element_type=jnp.float32)

            xlb = x_ltm.astype(BF)
            qw = jnp.dot(xlb, wqw_ref[...].astype(BF),
                         preferred_element_type=jnp.float32)
            sw = jax.lax.dot_general((qw * ISQ).astype(BF), contentb,
                                     (((1,), (1,)), ((), ())),
                                     preferred_element_type=jnp.float32)
            sw = sw + logv
            mw = jnp.max(sw, axis=1, keepdims=True)
            ew = jnp.exp(sw - mw)
            aw = (ew / jnp.sum(ew, axis=1, keepdims=True)).astype(BF)
            readw = jnp.dot(aw, contentb, preferred_element_type=jnp.float32)
            gw = jax.nn.sigmoid(jnp.dot(xlb, wgwr_ref[...].astype(BF),
                                        preferred_element_type=jnp.float32))
            xe = x_ltm + gw * jnp.dot(readw.astype(BF), wow_ref[...].astype(BF),
                                      preferred_element_type=jnp.float32)

            hnf_ref[st * TS:(st + 1) * TS, :] = _ln(
                xe, g0_ref[...], be0_ref[...]).astype(BF)
            for j in range(DT):
                xet_ref[j, st * TS:(st + 1) * TS, :] = \
                    xe[:, j * DTILE:(j + 1) * DTILE].astype(BF)

        _shift_store(hnc_ref, hnf_ref[...], 1)

    part = jnp.dot(hnc_ref[...], w0_ref[...].astype(BF),
                   preferred_element_type=jnp.float32)
    h1_ref[0] = (xet_ref[dt].astype(jnp.float32)
                 + jax.nn.gelu(part + b0_ref[...])).astype(BF)


def _mega_b(h1f_ref, lc_ref, wmc_ref, wmv_ref, w1_ref, b1_ref,
            g1_ref, be1_ref, png_ref, pnb_ref, wqww_ref, wvww_ref, wgww_ref,
            wk_ref, wv_ref, wg_ref,
            out_ref, slice_ref, wmc_out_ref, wmv_out_ref,
            hnc_ref, h1tt_ref, h2t_ref):
    dt = pl.program_id(1)

    @pl.when(dt == 0)
    def _():
        h1 = h1f_ref[0].astype(jnp.float32)
        hn = _ln(h1, g1_ref[...], be1_ref[...]).astype(BF)
        _shift_store(hnc_ref, hn, 2)
        for j in range(DT):
            h1tt_ref[j] = h1f_ref[0][:, j * DTILE:(j + 1) * DTILE]

    part = jnp.dot(hnc_ref[...], w1_ref[...].astype(BF),
                   preferred_element_type=jnp.float32)
    h2t_ref[dt] = h1tt_ref[dt].astype(jnp.float32) + jax.nn.gelu(part + b1_ref[...])

    @pl.when(dt == DT - 1)
    def _():
        h2 = jnp.concatenate([h2t_ref[j] for j in range(DT)], axis=1)
        o = _ln(h2, png_ref[...], pnb_ref[...])
        out_ref[0] = o
        pooled = jnp.mean(o, axis=0, keepdims=True)

        content = wmc_ref[0]
        valid = wmv_ref[0]
        pq = jnp.dot(pooled, wqww_ref[...], preferred_element_type=jnp.float32)
        ws = jax.lax.dot_general(pq, content, (((1,), (1,)), ((), ())),
                                 preferred_element_type=jnp.float32)
        mx = jnp.max(ws, axis=1, keepdims=True)
        iota_l = jax.lax.broadcasted_iota(jnp.int32, (1, NW), 1)
        slot = jnp.min(jnp.where(ws >= mx, iota_l, NW))
        mask_col = jax.lax.broadcasted_iota(jnp.int32, (NW, 1), 0) == slot
        wv_val = jnp.dot(pooled, wvww_ref[...],
                         preferred_element_type=jnp.float32)
        wg_val = jax.nn.sigmoid(jnp.dot(pooled, wgww_ref[...],
                                        preferred_element_type=jnp.float32))
        old = jnp.sum(jnp.where(mask_col, content, 0.0), axis=0, keepdims=True)
        newc = wg_val * wv_val + (1.0 - wg_val) * old
        wmc_out_ref[0] = jnp.where(mask_col, newc, content)
        wgs = wg_val[0, 0]
        wmv_out_ref[0] = jnp.where(iota_l == slot,
                                   jnp.maximum(valid, wgs), valid)

        ob = o.astype(BF)
        lc = lc_ref[0]
        lcb = lc.astype(BF)
        kx = jnp.dot(ob, wk_ref[...].astype(BF),
                     preferred_element_type=jnp.float32)
        vx = jnp.dot(ob, wv_ref[...].astype(BF),
                     preferred_element_type=jnp.float32)
        al = jax.lax.dot_general((kx * ISQ).astype(BF), lcb,
                                 (((1,), (1,)), ((), ())),
                                 preferred_element_type=jnp.float32).astype(BF)
        m = jnp.max(al, axis=1, keepdims=True)
        e = jnp.exp(al - m)
        rs = jnp.sum(e.astype(jnp.float32), axis=1, keepdims=True)
        gw = jax.nn.sigmoid(jnp.dot(ob, wg_ref[...].astype(BF),
                                    preferred_element_type=jnp.float32))
        wts = e * (gw / rs).astype(BF)
        ones = jnp.ones((S, 1), BF)
        wsum = jax.lax.dot_general(wts, ones, (((0,), (0,)), ((), ())),
                                   preferred_element_type=jnp.float32)
        vavg = jax.lax.dot_general(wts, vx.astype(BF), (((0,), (0,)), ((), ())),
                                   preferred_element_type=jnp.float32)
        vavg = vavg / (wsum + 1e-6)
        blend = jnp.clip(wsum, 0.0, 1.0)
        slice_ref[0] = lc * (1.0 - blend) + vavg * blend


def _row2d(a):
    return a.reshape(1, -1)


def kernel(x, cache, wm, Wq_ltm, Wo_ltm, Wg_ltm_r, Wk_ltm_w, Wv_ltm_w,
           Wg_ltm_w, Wq_wm, Wo_wm, Wg_wm_r, Wq_wm_w, Wv_wm_w, Wg_wm_w,
           conv0_w, conv1_w, conv0_b, conv1_b, ln0_g, ln0_b, ln1_g, ln1_b,
           pn_g, pn_b):
    wmc = wm[..., :DC]
    wmv = jnp.transpose(wm[..., DC:], (0, 2, 1))
    w0 = conv0_w.reshape(KS * D, D)
    w1 = conv1_w.reshape(KS * D, D)

    full = lambda *shape: pl.BlockSpec(shape, lambda b, dt: (0,) * len(shape))

    h1 = pl.pallas_call(
        _mega_a,
        grid=(B, DT),
        in_specs=[
            pl.BlockSpec((1, S, D), lambda b, dt: (b, 0, 0)),
            pl.BlockSpec((1, NTOT, DC), lambda b, dt: (b, 0, 0)),
            pl.BlockSpec((1, NW, DC), lambda b, dt: (b, 0, 0)),
            pl.BlockSpec((1, 1, NW), lambda b, dt: (b, 0, 0)),
            full(D, DC), full(DC, D), full(D, 1),
            full(D, DC), full(DC, D), full(D, 1),
            pl.BlockSpec((KS * D, DTILE), lambda b, dt: (0, dt)),
            pl.BlockSpec((1, DTILE), lambda b, dt: (0, dt)),
            full(1, D), full(1, D),
        ],
        out_specs=pl.BlockSpec((1, S, DTILE), lambda b, dt: (b, 0, dt)),
        out_shape=jax.ShapeDtypeStruct((B, S, D), BF),
        scratch_shapes=[
            pltpu.VMEM((NTOT, 2 * DC), BF),
            pltpu.VMEM((DT, S, DTILE), BF),
            pltpu.VMEM((S, D), BF),
            pltpu.VMEM((S, KS * D), BF),
        ],
        compiler_params=pltpu.CompilerParams(
            dimension_semantics=("parallel", "arbitrary")),
    )(x, cache, wmc, wmv, Wq_ltm, Wo_ltm, Wg_ltm_r, Wq_wm, Wo_wm, Wg_wm_r,
      w0, _row2d(conv0_b), _row2d(ln0_g), _row2d(ln0_b))

    output, new_slice, wmc_u, wmv_u = pl.pallas_call(
        _mega_b,
        grid=(B, DT),
        in_specs=[
            pl.BlockSpec((1, S, D), lambda b, dt: (b, 0, 0)),
            pl.BlockSpec((1, NS, DC), lambda b, dt: (b, LI, 0)),
            pl.BlockSpec((1, NW, DC), lambda b, dt: (b, 0, 0)),
            pl.BlockSpec((1, 1, NW), lambda b, dt: (b, 0, 0)),
            pl.BlockSpec((KS * D, DTILE), lambda b, dt: (0, dt)),
            pl.BlockSpec((1, DTILE), lambda b, dt: (0, dt)),
            full(1, D), full(1, D), full(1, D), full(1, D),
            full(D, DC), full(D, DC), full(D, 1),
            full(D, DC), full(D, DC), full(D, 1),
        ],
        out_specs=[
            pl.BlockSpec((1, S, D), lambda b, dt: (b, 0, 0)),
            pl.BlockSpec((1, NS, DC), lambda b, dt: (b, 0, 0)),
            pl.BlockSpec((1, NW, DC), lambda b, dt: (b, 0, 0)),
            pl.BlockSpec((1, 1, NW), lambda b, dt: (b, 0, 0)),
        ],
        out_shape=[
            jax.ShapeDtypeStruct((B, S, D), jnp.float32),
            jax.ShapeDtypeStruct((B, NS, DC), jnp.float32),
            jax.ShapeDtypeStruct((B, NW, DC), jnp.float32),
            jax.ShapeDtypeStruct((B, 1, NW), jnp.float32),
        ],
        scratch_shapes=[
            pltpu.VMEM((S, KS * D), BF),
            pltpu.VMEM((DT, S, DTILE), BF),
            pltpu.VMEM((DT, S, DTILE), jnp.float32),
        ],
        compiler_params=pltpu.CompilerParams(
            dimension_semantics=("parallel", "arbitrary")),
    )(h1, cache, wmc, wmv, w1, _row2d(conv1_b), _row2d(ln1_g),
      _row2d(ln1_b), _row2d(pn_g), _row2d(pn_b), Wq_wm_w, Wv_wm_w, Wg_wm_w,
      Wk_ltm_w, Wv_ltm_w, Wg_ltm_w)

    cache_u = jax.lax.dynamic_update_slice_in_dim(cache, new_slice,
                                                  LI * NS, axis=1)
    wm_u = jnp.concatenate([wmc_u, jnp.transpose(wmv_u, (0, 2, 1))], axis=-1)
    return (output, cache_u, wm_u)


# speedup vs baseline: 1.0749x; 1.0038x over previous
import jax
import jax.numpy as jnp
import numpy as np
from jax.experimental import pallas as pl
from jax.experimental.pallas import tpu as pltpu

B, S, D, DC, NS, NL, LI, NW, KS = 2, 1024, 1024, 64, 1024, 8, 3, 8, 5
NTOT = NL * NS
ISQ = float(1.0 / np.sqrt(DC))
TS = 256
DTILE = 256
DT = D // DTILE
BF = jnp.bfloat16


def _shift_store(hnc_ref, hn, dil):
    for k in range(KS):
        shift = (KS - 1 - k) * dil
        if shift:
            sh = jnp.concatenate(
                [jnp.zeros((shift, D), BF), hn[:S - shift]], axis=0)
        else:
            sh = hn
        hnc_ref[:, k * D:(k + 1) * D] = sh


def _ln(x, g, b):
    m = jnp.mean(x, axis=1, keepdims=True)
    v = jnp.mean((x - m) ** 2, axis=1, keepdims=True)
    return (x - m) * jax.lax.rsqrt(v + 1e-5) * g + b


def _mega_a(x_ref, cache_ref, wm_ref, wq_ref, wo_ref, wgr_ref,
            wqw_ref, wow_ref, wgwr_ref, w0_ref, b0_ref, g0_ref, be0_ref,
            h1_ref, cba_ref, xet_ref, hnf_ref, hnc_ref):
    dt = pl.program_id(1)

    @pl.when(dt == 0)
    def _():
        cba_ref[:, 0:DC] = cache_ref[0].astype(BF)
        il = jax.lax.broadcasted_iota(jnp.int32, (NTOT, DC), 1)
        cba_ref[:, DC:2 * DC] = jnp.where(il == 0, 1.0, 0.0).astype(BF)
        cb = cba_ref[:, 0:DC]
        content = wm_ref[0][:, 0:DC]
        contentb = content.astype(BF)
        vcol = wm_ref[0][:, DC:DC + 1]
        diag = (jax.lax.broadcasted_iota(jnp.int32, (NW, NW), 0)
                == jax.lax.broadcasted_iota(jnp.int32, (NW, NW), 1))
        vrow = jnp.sum(jnp.where(diag, jnp.broadcast_to(vcol, (NW, NW)), 0.0),
                       axis=0, keepdims=True)
        logv = jnp.log(vrow + 1e-6)

        for st in range(S // TS):
            x = x_ref[0, st * TS:(st + 1) * TS, :]
            xb = x.astype(BF)
            q = jnp.dot(xb, wq_ref[...].astype(BF),
                        preferred_element_type=jnp.float32)
            qb = (q * ISQ).astype(BF)
            logits = jax.lax.dot_general(
                qb, cb, (((1,), (1,)), ((), ())),
                preferred_element_type=jnp.float32).astype(BF)
            m = jnp.max(logits, axis=1, keepdims=True)
            e = jnp.exp(logits - m)
            ra = jnp.dot(e, cba_ref[...], preferred_element_type=jnp.float32)
            read = ra[:, 0:DC] / ra[:, DC:DC + 1]
            g = jax.nn.sigmoid(jnp.dot(xb, wgr_ref[...].astype(BF),
                                       preferred_element_type=jnp.float32))
            x_ltm = x + g * jnp.dot(read.astype(BF), wo_ref[...].astype(BF),
                                    preferred_element_type=jnp.float32)

            xlb = x_ltm.astype(BF)
            qw = jnp.dot(xlb, wqw_ref[...].astype(BF),
                         preferred_element_type=jnp.float32)
            sw = jax.lax.dot_general((qw * ISQ).astype(BF), contentb,
                                     (((1,), (1,)), ((), ())),
                                     preferred_element_type=jnp.float32)
            sw = sw + logv
            mw = jnp.max(sw, axis=1, keepdims=True)
            ew = jnp.exp(sw - mw)
            aw = (ew / jnp.sum(ew, axis=1, keepdims=True)).astype(BF)
            readw = jnp.dot(aw, contentb, preferred_element_type=jnp.float32)
            gw = jax.nn.sigmoid(jnp.dot(xlb, wgwr_ref[...].astype(BF),
                                        preferred_element_type=jnp.float32))
            xe = x_ltm + gw * jnp.dot(readw.astype(BF), wow_ref[...].astype(BF),
                                      preferred_element_type=jnp.float32)

            hnf_ref[st * TS:(st + 1) * TS, :] = _ln(
                xe, g0_ref[...], be0_ref[...]).astype(BF)
            for j in range(DT):
                xet_ref[j, st * TS:(st + 1) * TS, :] = \
                    xe[:, j * DTILE:(j + 1) * DTILE]

        _shift_store(hnc_ref, hnf_ref[...], 1)

    part = jnp.dot(hnc_ref[...], w0_ref[...].astype(BF),
                   preferred_element_type=jnp.float32)
    h1_ref[0] = (xet_ref[dt] + jax.nn.gelu(part + b0_ref[...])).astype(BF)


def _mega_b(h1f_ref, lc_ref, wm_ref, w1_ref, b1_ref,
            g1_ref, be1_ref, png_ref, pnb_ref, wqww_ref, wvww_ref, wgww_ref,
            wk_ref, wv_ref, wg_ref,
            out_ref, slice_ref, wm_out_ref,
            hnc_ref, h1tt_ref, h2t_ref):
    dt = pl.program_id(1)

    @pl.when(dt == 0)
    def _():
        h1 = h1f_ref[0].astype(jnp.float32)
        hn = _ln(h1, g1_ref[...], be1_ref[...]).astype(BF)
        _shift_store(hnc_ref, hn, 2)
        for j in range(DT):
            h1tt_ref[j] = h1f_ref[0][:, j * DTILE:(j + 1) * DTILE]

    part = jnp.dot(hnc_ref[...], w1_ref[...].astype(BF),
                   preferred_element_type=jnp.float32)
    h2t_ref[dt] = h1tt_ref[dt].astype(jnp.float32) + jax.nn.gelu(part + b1_ref[...])

    @pl.when(dt == DT - 1)
    def _():
        h2 = jnp.concatenate([h2t_ref[j] for j in range(DT)], axis=1)
        o = _ln(h2, png_ref[...], pnb_ref[...])
        out_ref[0] = o
        pooled = jnp.mean(o, axis=0, keepdims=True)

        content = wm_ref[0][:, 0:DC]
        vcol = wm_ref[0][:, DC:DC + 1]
        pq = jnp.dot(pooled, wqww_ref[...], preferred_element_type=jnp.float32)
        ws = jax.lax.dot_general(pq, content, (((1,), (1,)), ((), ())),
                                 preferred_element_type=jnp.float32)
        mx = jnp.max(ws, axis=1, keepdims=True)
        iota_l = jax.lax.broadcasted_iota(jnp.int32, (1, NW), 1)
        slot = jnp.min(jnp.where(ws >= mx, iota_l, NW))
        mask_col = jax.lax.broadcasted_iota(jnp.int32, (NW, 1), 0) == slot
        wv_val = jnp.dot(pooled, wvww_ref[...],
                         preferred_element_type=jnp.float32)
        wg_val = jax.nn.sigmoid(jnp.dot(pooled, wgww_ref[...],
                                        preferred_element_type=jnp.float32))
        old = jnp.sum(jnp.where(mask_col, content, 0.0), axis=0, keepdims=True)
        newc = wg_val * wv_val + (1.0 - wg_val) * old
        wgs = wg_val[0, 0]
        new_vcol = jnp.where(mask_col, jnp.maximum(vcol, wgs), vcol)
        wm_out_ref[0] = jnp.concatenate(
            [jnp.where(mask_col, newc, content), new_vcol], axis=1)

        ob = o.astype(BF)
        lc = lc_ref[0]
        lcb = lc.astype(BF)
        kx = jnp.dot(ob, wk_ref[...].astype(BF),
                     preferred_element_type=jnp.float32)
        vx = jnp.dot(ob, wv_ref[...].astype(BF),
                     preferred_element_type=jnp.float32)
        al = jax.lax.dot_general((kx * ISQ).astype(BF), lcb,
                                 (((1,), (1,)), ((), ())),
                                 preferred_element_type=jnp.float32).astype(BF)
        m = jnp.max(al, axis=1, keepdims=True)
        e = jnp.exp(al - m)
        rs = jnp.sum(e.astype(jnp.float32), axis=1, keepdims=True)
        gw = jax.nn.sigmoid(jnp.dot(ob, wg_ref[...].astype(BF),
                                    preferred_element_type=jnp.float32))
        wts = e * (gw / rs).astype(BF)
        ones = jnp.ones((S, 1), BF)
        wsum = jax.lax.dot_general(wts, ones, (((0,), (0,)), ((), ())),
                                   preferred_element_type=jnp.float32)
        vavg = jax.lax.dot_general(wts, vx.astype(BF), (((0,), (0,)), ((), ())),
                                   preferred_element_type=jnp.float32)
        vavg = vavg / (wsum + 1e-6)
        blend = jnp.clip(wsum, 0.0, 1.0)
        slice_ref[0] = lc * (1.0 - blend) + vavg * blend


def _row2d(a):
    return a.reshape(1, -1)


def kernel(x, cache, wm, Wq_ltm, Wo_ltm, Wg_ltm_r, Wk_ltm_w, Wv_ltm_w,
           Wg_ltm_w, Wq_wm, Wo_wm, Wg_wm_r, Wq_wm_w, Wv_wm_w, Wg_wm_w,
           conv0_w, conv1_w, conv0_b, conv1_b, ln0_g, ln0_b, ln1_g, ln1_b,
           pn_g, pn_b):
    w0 = conv0_w.reshape(KS * D, D)
    w1 = conv1_w.reshape(KS * D, D)

    full = lambda *shape: pl.BlockSpec(shape, lambda b, dt: (0,) * len(shape))

    h1 = pl.pallas_call(
        _mega_a,
        grid=(B, DT),
        in_specs=[
            pl.BlockSpec((1, S, D), lambda b, dt: (b, 0, 0)),
            pl.BlockSpec((1, NTOT, DC), lambda b, dt: (b, 0, 0)),
            pl.BlockSpec((1, NW, DC + 1), lambda b, dt: (b, 0, 0)),
            full(D, DC), full(DC, D), full(D, 1),
            full(D, DC), full(DC, D), full(D, 1),
            pl.BlockSpec((KS * D, DTILE), lambda b, dt: (0, dt)),
            pl.BlockSpec((1, DTILE), lambda b, dt: (0, dt)),
            full(1, D), full(1, D),
        ],
        out_specs=pl.BlockSpec((1, S, DTILE), lambda b, dt: (b, 0, dt)),
        out_shape=jax.ShapeDtypeStruct((B, S, D), BF),
        scratch_shapes=[
            pltpu.VMEM((NTOT, 2 * DC), BF),
            pltpu.VMEM((DT, S, DTILE), jnp.float32),
            pltpu.VMEM((S, D), BF),
            pltpu.VMEM((S, KS * D), BF),
        ],
        compiler_params=pltpu.CompilerParams(
            dimension_semantics=("parallel", "arbitrary")),
    )(x, cache, wm, Wq_ltm, Wo_ltm, Wg_ltm_r, Wq_wm, Wo_wm, Wg_wm_r,
      w0, _row2d(conv0_b), _row2d(ln0_g), _row2d(ln0_b))

    output, new_slice, wm_u = pl.pallas_call(
        _mega_b,
        grid=(B, DT),
        in_specs=[
            pl.BlockSpec((1, S, D), lambda b, dt: (b, 0, 0)),
            pl.BlockSpec((1, NS, DC), lambda b, dt: (b, LI, 0)),
            pl.BlockSpec((1, NW, DC + 1), lambda b, dt: (b, 0, 0)),
            pl.BlockSpec((KS * D, DTILE), lambda b, dt: (0, dt)),
            pl.BlockSpec((1, DTILE), lambda b, dt: (0, dt)),
            full(1, D), full(1, D), full(1, D), full(1, D),
            full(D, DC), full(D, DC), full(D, 1),
            full(D, DC), full(D, DC), full(D, 1),
        ],
        out_specs=[
            pl.BlockSpec((1, S, D), lambda b, dt: (b, 0, 0)),
            pl.BlockSpec((1, NS, DC), lambda b, dt: (b, 0, 0)),
            pl.BlockSpec((1, NW, DC + 1), lambda b, dt: (b, 0, 0)),
        ],
        out_shape=[
            jax.ShapeDtypeStruct((B, S, D), jnp.float32),
            jax.ShapeDtypeStruct((B, NS, DC), jnp.float32),
            jax.ShapeDtypeStruct((B, NW, DC + 1), jnp.float32),
        ],
        scratch_shapes=[
            pltpu.VMEM((S, KS * D), BF),
            pltpu.VMEM((DT, S, DTILE), BF),
            pltpu.VMEM((DT, S, DTILE), jnp.float32),
        ],
        compiler_params=pltpu.CompilerParams(
            dimension_semantics=("parallel", "arbitrary")),
    )(h1, cache, wm, w1, _row2d(conv1_b), _row2d(ln1_g),
      _row2d(ln1_b), _row2d(pn_g), _row2d(pn_b), Wq_wm_w, Wv_wm_w, Wg_wm_w,
      Wk_ltm_w, Wv_ltm_w, Wg_ltm_w)

    cache_u = jax.lax.dynamic_update_slice_in_dim(cache, new_slice,
                                                  LI * NS, axis=1)
    return (output, cache_u, wm_u)
